# baseline (speedup 1.0000x reference)
import functools

import jax
import jax.numpy as jnp
from jax import lax
from jax.experimental import pallas as pl
from jax.experimental.pallas import tpu as pltpu
from jax.experimental.pallas import tpu_sc as plsc

NC, NS = 2, 16
NW = NC * NS
CHUNK = 128


def _mesh():
    return plsc.VectorSubcoreMesh(core_axis_name="c", subcore_axis_name="s",
                                  num_cores=NC, num_subcores=NS)


_SC_PARAMS = pltpu.CompilerParams(use_tc_tiling_on_sc=False)


def _zero_fill(ref, rows, feat):
    z = jnp.zeros((16,), jnp.float32)

    @pl.loop(0, rows)
    def _(r):
        for f0 in range(feat // 16):
            ref[r, pl.ds(f0 * 16, 16)] = z


def _make_deg_kernel(n_pad, nb, extras):
    rows_per_s = n_pad // NS

    @functools.partial(
        pl.kernel,
        out_type=jax.ShapeDtypeStruct((NC, n_pad), jnp.float32),
        mesh=_mesh(),
        compiler_params=_SC_PARAMS,
        scratch_types=[
            pltpu.VMEM_SHARED((n_pad,), jnp.float32),
            pltpu.VMEM((nb + 1, CHUNK), jnp.int32),
            pltpu.VMEM((nb + 1, CHUNK), jnp.float32),
            pltpu.VMEM((rows_per_s,), jnp.float32),
            pltpu.SemaphoreType.DMA,
        ],
    )
    def deg_kernel(dst_hbm, ew_hbm, out_hbm, acc, dstb, ewb, zb, sem):
        c = lax.axis_index("c")
        s = lax.axis_index("s")
        wid = c * NS + s

        @pl.loop(0, rows_per_s, step=16)
        def _(i):
            zb[pl.ds(i, 16)] = jnp.zeros((16,), jnp.float32)

        pltpu.sync_copy(zb, acc.at[pl.ds(s * rows_per_s, rows_per_s)])
        pltpu.sync_copy(ew_hbm.at[pl.ds(wid * nb, nb)],
                        ewb.at[pl.ds(0, nb)])
        pltpu.sync_copy(dst_hbm.at[pl.ds(wid * nb, nb)],
                        dstb.at[pl.ds(0, nb)])

        @pl.when(wid < extras)
        def _():
            pltpu.sync_copy(ew_hbm.at[pl.ds(NW * nb + wid, 1)],
                            ewb.at[pl.ds(nb, 1)])
            pltpu.sync_copy(dst_hbm.at[pl.ds(NW * nb + wid, 1)],
                            dstb.at[pl.ds(nb, 1)])

        plsc.subcore_barrier()

        @pl.loop(0, nb)
        def _(j):
            pltpu.async_copy(ewb.at[j], acc.at[dstb.at[j]], sem, add=True)

        @pl.when(wid < extras)
        def _():
            pltpu.async_copy(ewb.at[nb], acc.at[dstb.at[nb]], sem, add=True)

        @pl.loop(0, nb)
        def _(j):
            pltpu.make_async_copy(ewb.at[j], acc.at[dstb.at[j]], sem).wait()

        @pl.when(wid < extras)
        def _():
            pltpu.make_async_copy(ewb.at[nb], acc.at[dstb.at[nb]],
                                  sem).wait()

        plsc.subcore_barrier()
        pltpu.sync_copy(acc.at[pl.ds(s * rows_per_s, rows_per_s)],
                        out_hbm.at[c, pl.ds(s * rows_per_s, rows_per_s)])

    return deg_kernel


def _make_edge_kernel(n, n_pad, nb, extras, feat):
    rows_per_s = n_pad // NS
    stage_rows = n // NS

    @functools.partial(
        pl.kernel,
        out_type=jax.ShapeDtypeStruct((NC, n_pad, feat), jnp.float32),
        mesh=_mesh(),
        compiler_params=_SC_PARAMS,
        scratch_types=[
            pltpu.VMEM_SHARED((n_pad, feat), jnp.float32),
            pltpu.VMEM_SHARED((n, feat), jnp.float32),
            pltpu.VMEM((nb + 1, CHUNK), jnp.int32),
            pltpu.VMEM((nb + 1, CHUNK), jnp.int32),
            pltpu.VMEM((nb + 1, CHUNK), jnp.float32),
            pltpu.VMEM((CHUNK, feat), jnp.float32),
            pltpu.VMEM((CHUNK, feat), jnp.float32),
            pltpu.VMEM((rows_per_s, feat), jnp.float32),
            pltpu.SemaphoreType.DMA,
            pltpu.SemaphoreType.DMA,
            pltpu.SemaphoreType.DMA,
            pltpu.SemaphoreType.DMA,
        ],
    )
    def edge_kernel(y_hbm, src_hbm, dst_hbm, ew_hbm, out_hbm,
                    acc, ysp, srcb, dstb, ewb, gb0, gb1, zb,
                    sem0, sem1, ssem0, ssem1):
        c = lax.axis_index("c")
        s = lax.axis_index("s")
        wid = c * NS + s

        pltpu.async_copy(y_hbm.at[pl.ds(s * stage_rows, stage_rows)],
                         ysp.at[pl.ds(s * stage_rows, stage_rows)], sem0)

        pltpu.async_copy(src_hbm.at[pl.ds(wid * nb, nb)],
                         srcb.at[pl.ds(0, nb)], sem1)
        pltpu.async_copy(dst_hbm.at[pl.ds(wid * nb, nb)],
                         dstb.at[pl.ds(0, nb)], ssem0)
        pltpu.async_copy(ew_hbm.at[pl.ds(wid * nb, nb)],
                         ewb.at[pl.ds(0, nb)], ssem1)

        @pl.when(wid < extras)
        def _():
            pltpu.sync_copy(src_hbm.at[pl.ds(NW * nb + wid, 1)],
                            srcb.at[pl.ds(nb, 1)])
            pltpu.sync_copy(dst_hbm.at[pl.ds(NW * nb + wid, 1)],
                            dstb.at[pl.ds(nb, 1)])
            pltpu.sync_copy(ew_hbm.at[pl.ds(NW * nb + wid, 1)],
                            ewb.at[pl.ds(nb, 1)])

        _zero_fill(zb, rows_per_s, feat)
        pltpu.sync_copy(zb, acc.at[pl.ds(s * rows_per_s, rows_per_s)])
        pltpu.make_async_copy(src_hbm.at[pl.ds(0, nb)],
                              srcb.at[pl.ds(0, nb)], sem1).wait()
        pltpu.make_async_copy(dst_hbm.at[pl.ds(0, nb)],
                              dstb.at[pl.ds(0, nb)], ssem0).wait()
        pltpu.make_async_copy(ew_hbm.at[pl.ds(0, nb)],
                              ewb.at[pl.ds(0, nb)], ssem1).wait()
        pltpu.make_async_copy(y_hbm.at[pl.ds(0, stage_rows)],
                              ysp.at[pl.ds(0, stage_rows)], sem0).wait()
        plsc.subcore_barrier()

        def gstart(j, gb, sem):
            pltpu.async_copy(ysp.at[srcb.at[j]], gb, sem)

        def gwait(j, gb, sem):
            pltpu.make_async_copy(ysp.at[srcb.at[j]], gb, sem).wait()

        def sstart(j, gb, sem):
            pltpu.async_copy(gb, acc.at[dstb.at[j]], sem, add=True)

        def swait(j, gb, sem):
            pltpu.make_async_copy(gb, acc.at[dstb.at[j]], sem).wait()

        def mult(j, gb):
            @pl.loop(0, CHUNK, step=16)
            def _(e0):
                ewv = ewb[j, pl.ds(e0, 16)]
                for i in range(16):
                    w = ewv[i]
                    for f0 in range(feat // 16):
                        sl = pl.ds(f0 * 16, 16)
                        gb[e0 + i, sl] = gb[e0 + i, sl] * w

        gstart(0, gb0, sem0)

        @pl.loop(0, nb, step=2)
        def _(j):
            @pl.when(j > 0)
            def _():
                swait(j - 1, gb1, ssem1)

            gstart(j + 1, gb1, sem1)
            gwait(j, gb0, sem0)
            mult(j, gb0)
            sstart(j, gb0, ssem0)

            @pl.when(j + 2 < nb)
            def _():
                swait(j, gb0, ssem0)
                gstart(j + 2, gb0, sem0)

            gwait(j + 1, gb1, sem1)
            mult(j + 1, gb1)
            sstart(j + 1, gb1, ssem1)

        swait(nb - 2, gb0, ssem0)
        swait(nb - 1, gb1, ssem1)

        @pl.when(wid < extras)
        def _():
            pltpu.sync_copy(ysp.at[srcb.at[nb]], gb0)
            mult(nb, gb0)
            pltpu.sync_copy(gb0, acc.at[dstb.at[nb]], add=True)

        plsc.subcore_barrier()
        pltpu.sync_copy(acc.at[pl.ds(s * rows_per_s, rows_per_s)],
                        out_hbm.at[c, pl.ds(s * rows_per_s, rows_per_s)])

    return edge_kernel


def _tc_xw(x, w1):
    n, _ = x.shape
    f = w1.shape[1]

    def body(x_ref, w_ref, o_ref):
        o_ref[...] = jnp.dot(x_ref[...], w_ref[...],
                             preferred_element_type=jnp.float32)

    return pl.pallas_call(
        body, out_shape=jax.ShapeDtypeStruct((n, f), jnp.float32))(x, w1)


def _tc_prep(degp, xw):
    n, f = xw.shape

    def body(degp_ref, xw_ref, dis_ref, y_ref):
        deg = (degp_ref[0] + degp_ref[1])[:n].reshape(n, 1) + 1.0
        dis = lax.rsqrt(deg)
        dis_ref[...] = dis
        y_ref[...] = xw_ref[...] * dis

    return pl.pallas_call(
        body,
        out_shape=(jax.ShapeDtypeStruct((n, 1), jnp.float32),
                   jax.ShapeDtypeStruct((n, f), jnp.float32)))(degp, xw)


def _tc_mid(accp, y1, dis, b1, w2):
    n, f1 = y1.shape
    f2 = w2.shape[1]

    def body(accp_ref, y1_ref, dis_ref, b1_ref, w2_ref, y2_ref):
        sacc = accp_ref[0, :n, :] + accp_ref[1, :n, :]
        t = dis_ref[...] * (sacc + y1_ref[...]) + b1_ref[...]
        h = jnp.where(t >= 0, t, 0.01 * t)
        y2_ref[...] = jnp.dot(h, w2_ref[...],
                              preferred_element_type=jnp.float32) * dis_ref[...]

    return pl.pallas_call(
        body,
        out_shape=jax.ShapeDtypeStruct((n, f2), jnp.float32))(
            accp, y1, dis, b1, w2)


def _tc_final(accp, y2, dis, b2, wf, bf):
    n, f2 = y2.shape

    def body(accp_ref, y2_ref, dis_ref, b2_ref, wf_ref, bf_ref, o_ref):
        sacc = accp_ref[0, :n, :] + accp_ref[1, :n, :]
        t = dis_ref[...] * (sacc + y2_ref[...]) + b2_ref[...]
        x2 = jnp.where(t >= 0, t, 0.01 * t)
        o_ref[...] = jnp.dot(x2, wf_ref[...],
                             preferred_element_type=jnp.float32) + bf_ref[...]

    return pl.pallas_call(
        body,
        out_shape=jax.ShapeDtypeStruct((n, 1), jnp.float32))(
            accp, y2, dis, b2, wf, bf)


def kernel(x, edge_index, edge_weight, W1, b1, W2, b2, Wf, bf):
    n, _ = x.shape
    e = edge_index.shape[1]
    f1 = W1.shape[1]
    f2 = W2.shape[1]
    n_pad = ((n + 8 * NS - 1) // (8 * NS)) * (8 * NS)
    rows = e // CHUNK
    nb = (rows // NW) & ~1
    extras = rows - nb * NW

    src = edge_index[0].reshape(rows, CHUNK)
    dst = edge_index[1].reshape(rows, CHUNK)
    ew = edge_weight.reshape(rows, CHUNK)

    degp = _make_deg_kernel(n_pad, nb, extras)(dst, ew)
    xw = _tc_xw(x, W1)
    dis, y1 = _tc_prep(degp, xw)
    acc1 = _make_edge_kernel(n, n_pad, nb, extras, f1)(y1, src, dst, ew)
    y2 = _tc_mid(acc1, y1, dis, b1, W2)
    acc2 = _make_edge_kernel(n, n_pad, nb, extras, f2)(y2, src, dst, ew)
    return _tc_final(acc2, y2, dis, b2, Wf, bf)


# speedup vs baseline: 1.0231x; 1.0231x over previous
import functools

import jax
import jax.numpy as jnp
from jax import lax
from jax.experimental import pallas as pl
from jax.experimental.pallas import tpu as pltpu
from jax.experimental.pallas import tpu_sc as plsc

NC, NS = 2, 16
NW = NC * NS
CHUNK = 128


def _mesh():
    return plsc.VectorSubcoreMesh(core_axis_name="c", subcore_axis_name="s",
                                  num_cores=NC, num_subcores=NS)


_SC_PARAMS = pltpu.CompilerParams(use_tc_tiling_on_sc=False)


def _zero_fill(ref, rows, feat):
    z = jnp.zeros((16,), jnp.float32)

    @pl.loop(0, rows)
    def _(r):
        for f0 in range(feat // 16):
            ref[r, pl.ds(f0 * 16, 16)] = z


def _make_deg_kernel(n_pad, nb, extras):
    rows_per_s = n_pad // NS

    @functools.partial(
        pl.kernel,
        out_type=jax.ShapeDtypeStruct((NC, n_pad), jnp.float32),
        mesh=_mesh(),
        compiler_params=_SC_PARAMS,
        scratch_types=[
            pltpu.VMEM_SHARED((n_pad,), jnp.float32),
            pltpu.VMEM((nb + 1, CHUNK), jnp.int32),
            pltpu.VMEM((nb + 1, CHUNK), jnp.float32),
            pltpu.VMEM((rows_per_s,), jnp.float32),
            pltpu.SemaphoreType.DMA,
        ],
    )
    def deg_kernel(dst_hbm, ew_hbm, out_hbm, acc, dstb, ewb, zb, sem):
        c = lax.axis_index("c")
        s = lax.axis_index("s")
        wid = c * NS + s

        @pl.loop(0, rows_per_s, step=16)
        def _(i):
            zb[pl.ds(i, 16)] = jnp.zeros((16,), jnp.float32)

        pltpu.sync_copy(zb, acc.at[pl.ds(s * rows_per_s, rows_per_s)])
        pltpu.sync_copy(ew_hbm.at[pl.ds(wid * nb, nb)],
                        ewb.at[pl.ds(0, nb)])
        pltpu.sync_copy(dst_hbm.at[pl.ds(wid * nb, nb)],
                        dstb.at[pl.ds(0, nb)])

        @pl.when(wid < extras)
        def _():
            pltpu.sync_copy(ew_hbm.at[pl.ds(NW * nb + wid, 1)],
                            ewb.at[pl.ds(nb, 1)])
            pltpu.sync_copy(dst_hbm.at[pl.ds(NW * nb + wid, 1)],
                            dstb.at[pl.ds(nb, 1)])

        plsc.subcore_barrier()

        @pl.loop(0, nb)
        def _(j):
            pltpu.async_copy(ewb.at[j], acc.at[dstb.at[j]], sem, add=True)

        @pl.when(wid < extras)
        def _():
            pltpu.async_copy(ewb.at[nb], acc.at[dstb.at[nb]], sem, add=True)

        @pl.loop(0, nb)
        def _(j):
            pltpu.make_async_copy(ewb.at[j], acc.at[dstb.at[j]], sem).wait()

        @pl.when(wid < extras)
        def _():
            pltpu.make_async_copy(ewb.at[nb], acc.at[dstb.at[nb]],
                                  sem).wait()

        plsc.subcore_barrier()
        pltpu.sync_copy(acc.at[pl.ds(s * rows_per_s, rows_per_s)],
                        out_hbm.at[c, pl.ds(s * rows_per_s, rows_per_s)])

    return deg_kernel


def _make_edge_kernel(n, n_pad, nb, extras, feat):
    rows_per_s = n_pad // NS
    stage_rows = n // NS

    @functools.partial(
        pl.kernel,
        out_type=jax.ShapeDtypeStruct((NC, n_pad, feat), jnp.float32),
        mesh=_mesh(),
        compiler_params=_SC_PARAMS,
        scratch_types=[
            pltpu.VMEM_SHARED((n_pad, feat), jnp.float32),
            pltpu.VMEM_SHARED((n, feat), jnp.float32),
            pltpu.VMEM((nb + 1, CHUNK), jnp.int32),
            pltpu.VMEM((nb + 1, CHUNK), jnp.int32),
            pltpu.VMEM((nb + 1, CHUNK), jnp.float32),
            pltpu.VMEM((CHUNK, feat), jnp.float32),
            pltpu.VMEM((CHUNK, feat), jnp.float32),
            pltpu.VMEM((rows_per_s, feat), jnp.float32),
            pltpu.SemaphoreType.DMA,
            pltpu.SemaphoreType.DMA,
            pltpu.SemaphoreType.DMA,
            pltpu.SemaphoreType.DMA,
        ],
    )
    def edge_kernel(y_hbm, src_hbm, dst_hbm, ew_hbm, out_hbm,
                    acc, ysp, srcb, dstb, ewb, gb0, gb1, zb,
                    sem0, sem1, ssem0, ssem1):
        c = lax.axis_index("c")
        s = lax.axis_index("s")
        wid = c * NS + s

        pltpu.async_copy(y_hbm.at[pl.ds(s * stage_rows, stage_rows)],
                         ysp.at[pl.ds(s * stage_rows, stage_rows)], sem0)

        pltpu.async_copy(src_hbm.at[pl.ds(wid * nb, nb)],
                         srcb.at[pl.ds(0, nb)], sem1)
        pltpu.async_copy(dst_hbm.at[pl.ds(wid * nb, nb)],
                         dstb.at[pl.ds(0, nb)], ssem0)
        pltpu.async_copy(ew_hbm.at[pl.ds(wid * nb, nb)],
                         ewb.at[pl.ds(0, nb)], ssem1)

        @pl.when(wid < extras)
        def _():
            pltpu.sync_copy(src_hbm.at[pl.ds(NW * nb + wid, 1)],
                            srcb.at[pl.ds(nb, 1)])
            pltpu.sync_copy(dst_hbm.at[pl.ds(NW * nb + wid, 1)],
                            dstb.at[pl.ds(nb, 1)])
            pltpu.sync_copy(ew_hbm.at[pl.ds(NW * nb + wid, 1)],
                            ewb.at[pl.ds(nb, 1)])

        _zero_fill(zb, rows_per_s, feat)
        pltpu.sync_copy(zb, acc.at[pl.ds(s * rows_per_s, rows_per_s)])
        pltpu.make_async_copy(src_hbm.at[pl.ds(0, nb)],
                              srcb.at[pl.ds(0, nb)], sem1).wait()
        pltpu.make_async_copy(dst_hbm.at[pl.ds(0, nb)],
                              dstb.at[pl.ds(0, nb)], ssem0).wait()
        pltpu.make_async_copy(ew_hbm.at[pl.ds(0, nb)],
                              ewb.at[pl.ds(0, nb)], ssem1).wait()
        pltpu.make_async_copy(y_hbm.at[pl.ds(0, stage_rows)],
                              ysp.at[pl.ds(0, stage_rows)], sem0).wait()
        plsc.subcore_barrier()

        def gstart(j, gb, sem):
            pltpu.async_copy(ysp.at[srcb.at[j]], gb, sem)

        def gwait(j, gb, sem):
            pltpu.make_async_copy(ysp.at[srcb.at[j]], gb, sem).wait()

        def sstart(j, gb, sem):
            pltpu.async_copy(gb, acc.at[dstb.at[j]], sem, add=True)

        def swait(j, gb, sem):
            pltpu.make_async_copy(gb, acc.at[dstb.at[j]], sem).wait()

        def mult(j, gb):
            @pl.loop(0, CHUNK, step=16)
            def _(e0):
                ewv = ewb[j, pl.ds(e0, 16)]
                for i in range(16):
                    w = ewv[i]
                    for f0 in range(feat // 16):
                        sl = pl.ds(f0 * 16, 16)
                        gb[e0 + i, sl] = gb[e0 + i, sl] * w

        gstart(0, gb0, sem0)

        @pl.loop(0, nb, step=2)
        def _(j):
            @pl.when(j > 0)
            def _():
                swait(j - 1, gb1, ssem1)

            gstart(j + 1, gb1, sem1)
            gwait(j, gb0, sem0)
            mult(j, gb0)
            sstart(j, gb0, ssem0)

            @pl.when(j + 2 < nb)
            def _():
                swait(j, gb0, ssem0)
                gstart(j + 2, gb0, sem0)

            gwait(j + 1, gb1, sem1)
            mult(j + 1, gb1)
            sstart(j + 1, gb1, ssem1)

        swait(nb - 2, gb0, ssem0)
        swait(nb - 1, gb1, ssem1)

        @pl.when(wid < extras)
        def _():
            pltpu.sync_copy(ysp.at[srcb.at[nb]], gb0)
            mult(nb, gb0)
            pltpu.sync_copy(gb0, acc.at[dstb.at[nb]], add=True)

        plsc.subcore_barrier()
        pltpu.sync_copy(acc.at[pl.ds(s * rows_per_s, rows_per_s)],
                        out_hbm.at[c, pl.ds(s * rows_per_s, rows_per_s)])

    return edge_kernel


def _tc_xw(x, w1):
    n, _ = x.shape
    f = w1.shape[1]

    def body(x_ref, w_ref, o_ref):
        o_ref[...] = jnp.dot(x_ref[...], w_ref[...],
                             preferred_element_type=jnp.float32)

    return pl.pallas_call(
        body, out_shape=jax.ShapeDtypeStruct((n, f), jnp.float32))(x, w1)


def _dis_col(degp_ref, n):
    deg = (degp_ref[0] + degp_ref[1])[:n].reshape(n, 1) + 1.0
    return lax.rsqrt(deg)


def _tc_prep(degp, xw):
    n, f = xw.shape

    def body(degp_ref, xw_ref, y_ref):
        y_ref[...] = xw_ref[...] * _dis_col(degp_ref, n)

    return pl.pallas_call(
        body, out_shape=jax.ShapeDtypeStruct((n, f), jnp.float32))(degp, xw)


def _tc_mid(accp, y1, degp, b1, w2):
    n, f1 = y1.shape
    f2 = w2.shape[1]

    def body(accp_ref, y1_ref, degp_ref, b1_ref, w2_ref, y2_ref):
        dis = _dis_col(degp_ref, n)
        sacc = accp_ref[0, :n, :] + accp_ref[1, :n, :]
        t = dis * (sacc + y1_ref[...]) + b1_ref[...]
        h = jnp.where(t >= 0, t, 0.01 * t)
        y2_ref[...] = jnp.dot(h, w2_ref[...],
                              preferred_element_type=jnp.float32) * dis

    return pl.pallas_call(
        body,
        out_shape=jax.ShapeDtypeStruct((n, f2), jnp.float32))(
            accp, y1, degp, b1, w2)


def _tc_final(accp, y2, degp, b2, wf, bf):
    n, f2 = y2.shape

    def body(accp_ref, y2_ref, degp_ref, b2_ref, wf_ref, bf_ref, o_ref):
        dis = _dis_col(degp_ref, n)
        sacc = accp_ref[0, :n, :] + accp_ref[1, :n, :]
        t = dis * (sacc + y2_ref[...]) + b2_ref[...]
        x2 = jnp.where(t >= 0, t, 0.01 * t)
        o_ref[...] = jnp.dot(x2, wf_ref[...],
                             preferred_element_type=jnp.float32) + bf_ref[...]

    return pl.pallas_call(
        body,
        out_shape=jax.ShapeDtypeStruct((n, 1), jnp.float32))(
            accp, y2, degp, b2, wf, bf)


def kernel(x, edge_index, edge_weight, W1, b1, W2, b2, Wf, bf):
    n, _ = x.shape
    e = edge_index.shape[1]
    f1 = W1.shape[1]
    f2 = W2.shape[1]
    n_pad = ((n + 8 * NS - 1) // (8 * NS)) * (8 * NS)
    rows = e // CHUNK
    nb = (rows // NW) & ~1
    extras = rows - nb * NW

    src = edge_index[0].reshape(rows, CHUNK)
    dst = edge_index[1].reshape(rows, CHUNK)
    ew = edge_weight.reshape(rows, CHUNK)

    degp = _make_deg_kernel(n_pad, nb, extras)(dst, ew)
    xw = _tc_xw(x, W1)
    y1 = _tc_prep(degp, xw)
    acc1 = _make_edge_kernel(n, n_pad, nb, extras, f1)(y1, src, dst, ew)
    y2 = _tc_mid(acc1, y1, degp, b1, W2)
    acc2 = _make_edge_kernel(n, n_pad, nb, extras, f2)(y2, src, dst, ew)
    return _tc_final(acc2, y2, degp, b2, Wf, bf)
